# baseline (speedup 1.0000x reference)
import functools

import jax
import jax.numpy as jnp
from jax import lax
from jax.experimental import pallas as pl
from jax.experimental.pallas import tpu as pltpu
from jax.experimental.pallas import tpu_sc as plsc

K = 8
NIN = 10
D = 128
T = 262144

PK = K * K
D2 = 2 * D
TP = T // 2

NC = 2
NS = 16
NW = NC * NS
TOK_PER_W = T // NW
PAIR_PER_W = TOK_PER_W // 2
PCHUNK = 128
NPCH = PAIR_PER_W // PCHUNK

_sc_mesh = plsc.VectorSubcoreMesh(
    core_axis_name="c", subcore_axis_name="s", num_cores=NC, num_subcores=NS
)

NBUF = 2
LAG = 1
VL = 16


@functools.partial(
    pl.kernel,
    out_type=jax.ShapeDtypeStruct((TP, 2, D), jnp.float32),
    mesh=_sc_mesh,
    scratch_types=[
        pltpu.VMEM((TOK_PER_W,), jnp.int32),
        pltpu.VMEM((NPCH, PCHUNK), jnp.int32),
        pltpu.VMEM((K, VL), jnp.float32),
        pltpu.VMEM((NIN, D), jnp.float32),
        pltpu.VMEM((D,), jnp.float32),
        pltpu.VMEM((PK, 2, D), jnp.float32),
        pltpu.VMEM_SHARED((NS * PK, 2, D), jnp.float32),
        [pltpu.VMEM((PCHUNK, 2, D), jnp.float32)] * NBUF,
        pltpu.SemaphoreType.DMA,
        [pltpu.SemaphoreType.DMA] * NBUF,
        [pltpu.SemaphoreType.DMA] * NBUF,
    ],
)
def _sc_embed(table_hbm, w_hbm, b_hbm, tok_hbm, out_hbm,
              tokr, idx2, tab_v, w_v, b_v, pv2, pshared, rows,
              isem, gsem, wsem):
    sid = lax.axis_index("s")
    wid = sid * NC + lax.axis_index("c")
    base = wid * PAIR_PER_W

    idx_cp = pltpu.async_copy(tok_hbm.at[wid], tokr, isem)
    pltpu.sync_copy(table_hbm, tab_v)
    pltpu.sync_copy(w_hbm, w_v)
    pltpu.sync_copy(b_hbm, b_v)

    tab = []
    for k in range(K):
        trow = tab_v[k, pl.ds(0, VL)]
        tab.append([trow[i] for i in range(NIN)])
    accs = []
    for k in range(K):
        row = []
        for c in range(D // VL):
            acc = b_v[pl.ds(c * VL, VL)]
            for i in range(NIN):
                acc = acc + tab[k][i] * w_v[i, pl.ds(c * VL, VL)]
            row.append(acc)
        accs.append(row)

    for a in range(K):
        for b2 in range(K):
            for c in range(D // VL):
                pv2[a * K + b2, 0, pl.ds(c * VL, VL)] = accs[a][c]
                pv2[a * K + b2, 1, pl.ds(c * VL, VL)] = accs[b2][c]

    psrc = pshared.at[pl.ds(sid * PK, PK)]
    plsc.subcore_barrier()
    pltpu.sync_copy(pv2, psrc)
    plsc.subcore_barrier()
    idx_cp.wait()

    iot = lax.iota(jnp.int32, VL)
    c_shift = (iot + 1) & (VL - 1)
    c_lo = (iot * 2) & (VL - 1)
    c_hi = (iot * 2 - VL) & (VL - 1)
    _dnums = lax.GatherDimensionNumbers(
        offset_dims=(), collapsed_slice_dims=(0,), start_index_map=(0,)
    )

    def lane_gather(v, idx):
        return lax.gather(
            v, idx[:, None], dimension_numbers=_dnums, slice_sizes=(1,),
            mode=lax.GatherScatterMode.PROMISE_IN_BOUNDS,
        )

    def fuse(i, carry):
        v0 = tokr[pl.ds(i * 2 * VL, VL)]
        v1 = tokr[pl.ds(i * 2 * VL + VL, VL)]
        w0 = v0 * K + lane_gather(v0, c_shift)
        w1 = v1 * K + lane_gather(v1, c_shift)
        idxv = jnp.where(iot < K, lane_gather(w0, c_lo), lane_gather(w1, c_hi))
        idx2[i // (PCHUNK // VL), pl.ds((i % (PCHUNK // VL)) * VL, VL)] = idxv
        return carry

    lax.fori_loop(0, PAIR_PER_W // VL, fuse, 0)

    gd = [None] * NPCH
    wd = [None] * NPCH

    def write_back(i):
        b = i % NBUF
        gd[i].wait()
        wd[i] = pltpu.async_copy(
            rows[b], out_hbm.at[pl.ds(base + i * PCHUNK, PCHUNK)], wsem[b]
        )

    for j in range(NPCH):
        b = j % NBUF
        if j >= NBUF:
            wd[j - NBUF].wait()
        gd[j] = pltpu.async_copy(psrc.at[idx2.at[j]], rows[b], gsem[b])
        if j >= LAG:
            write_back(j - LAG)
    for i in range(NPCH - LAG, NPCH):
        write_back(i)
    for i in range(NPCH - NBUF, NPCH):
        wd[i].wait()


def kernel(tokens, table, W, b):
    tok2 = tokens.astype(jnp.int32).reshape(NW, TOK_PER_W)
    tab16 = jnp.pad(table, ((0, 0), (0, VL - NIN)))
    return _sc_embed(tab16, W, b, tok2).reshape(T, D)


# speedup vs baseline: 1.0783x; 1.0783x over previous
import functools

import jax
import jax.numpy as jnp
from jax import lax
from jax.experimental import pallas as pl
from jax.experimental.pallas import tpu as pltpu
from jax.experimental.pallas import tpu_sc as plsc

K = 8
NIN = 10
D = 128
T = 262144

NC = 2
NS = 16
NW = NC * NS
TOK_PER_W = T // NW
CHUNK = 128
NCHUNK = TOK_PER_W // CHUNK

_sc_mesh = plsc.VectorSubcoreMesh(
    core_axis_name="c", subcore_axis_name="s", num_cores=NC, num_subcores=NS
)

NBUF = 6
LAG = 3
VL = 16


@functools.partial(
    pl.kernel,
    out_type=jax.ShapeDtypeStruct((T, D), jnp.float32),
    mesh=_sc_mesh,
    scratch_types=[
        pltpu.VMEM((NCHUNK, CHUNK), jnp.int32),
        pltpu.VMEM((K, VL), jnp.float32),
        pltpu.VMEM((NIN, D), jnp.float32),
        pltpu.VMEM((D,), jnp.float32),
        pltpu.VMEM((K, D), jnp.float32),
        pltpu.VMEM_SHARED((NS * K, D), jnp.float32),
        [pltpu.VMEM((CHUNK, D), jnp.float32)] * NBUF,
        pltpu.SemaphoreType.DMA,
        [pltpu.SemaphoreType.DMA] * NBUF,
        [pltpu.SemaphoreType.DMA] * NBUF,
    ],
)
def _sc_embed(table_hbm, w_hbm, b_hbm, tok_hbm, out_hbm,
              idx_v, tab_v, w_v, b_v, pv, pshared, rows, isem, gsem, wsem):
    sid = lax.axis_index("s")
    wid = sid * NC + lax.axis_index("c")
    base = wid * TOK_PER_W

    idx_cp = pltpu.async_copy(tok_hbm.at[wid], idx_v, isem)
    pltpu.sync_copy(table_hbm, tab_v)
    pltpu.sync_copy(w_hbm, w_v)
    pltpu.sync_copy(b_hbm, b_v)

    tab = []
    for k in range(K):
        trow = tab_v[k, pl.ds(0, VL)]
        tab.append([trow[i] for i in range(NIN)])
    for c in range(D // VL):
        bvec = b_v[pl.ds(c * VL, VL)]
        wcol = [w_v[i, pl.ds(c * VL, VL)] for i in range(NIN)]
        for k in range(K):
            acc = bvec
            for i in range(NIN):
                acc = acc + tab[k][i] * wcol[i]
            pv[k, pl.ds(c * VL, VL)] = acc
    psrc = pshared.at[pl.ds(sid * K, K)]
    plsc.subcore_barrier()
    pltpu.sync_copy(pv, psrc)
    plsc.subcore_barrier()
    idx_cp.wait()

    gd = [None] * NCHUNK
    wd = [None] * NCHUNK

    def write_back(i):
        b = i % NBUF
        gd[i].wait()
        wd[i] = pltpu.async_copy(
            rows[b], out_hbm.at[pl.ds(base + i * CHUNK, CHUNK)], wsem[b]
        )

    for j in range(NCHUNK):
        b = j % NBUF
        if j >= NBUF:
            wd[j - NBUF].wait()
        gd[j] = pltpu.async_copy(psrc.at[idx_v.at[j]], rows[b], gsem[b])
        if j >= LAG:
            write_back(j - LAG)
    for i in range(NCHUNK - LAG, NCHUNK):
        write_back(i)
    for i in range(NCHUNK - NBUF, NCHUNK):
        wd[i].wait()


def kernel(tokens, table, W, b):
    tok3 = tokens.astype(jnp.int32).reshape(NW, NCHUNK, CHUNK)
    tab16 = jnp.pad(table, ((0, 0), (0, VL - NIN)))
    return _sc_embed(tab16, W, b, tok3)


# speedup vs baseline: 1.0799x; 1.0015x over previous
import functools

import jax
import jax.numpy as jnp
from jax import lax
from jax.experimental import pallas as pl
from jax.experimental.pallas import tpu as pltpu
from jax.experimental.pallas import tpu_sc as plsc

K = 8
NIN = 10
D = 128
T = 262144

NC = 2
NS = 16
NW = NC * NS
TOK_PER_W = T // NW
CHUNK = 128
NCHUNK = TOK_PER_W // CHUNK

_sc_mesh = plsc.VectorSubcoreMesh(
    core_axis_name="c", subcore_axis_name="s", num_cores=NC, num_subcores=NS
)

NBUF = 4
LAG = 2
VL = 16


@functools.partial(
    pl.kernel,
    out_type=jax.ShapeDtypeStruct((T, D), jnp.float32),
    mesh=_sc_mesh,
    scratch_types=[
        pltpu.VMEM((NCHUNK, CHUNK), jnp.int32),
        pltpu.VMEM((K, VL), jnp.float32),
        pltpu.VMEM((NIN, D), jnp.float32),
        pltpu.VMEM((D,), jnp.float32),
        pltpu.VMEM((K, D), jnp.float32),
        pltpu.VMEM_SHARED((NS * K, D), jnp.float32),
        [pltpu.VMEM((CHUNK, D), jnp.float32)] * NBUF,
        pltpu.SemaphoreType.DMA,
        [pltpu.SemaphoreType.DMA] * NBUF,
        [pltpu.SemaphoreType.DMA] * NBUF,
    ],
)
def _sc_embed(table_hbm, w_hbm, b_hbm, tok_hbm, out_hbm,
              idx_v, tab_v, w_v, b_v, pv, pshared, rows, isem, gsem, wsem):
    sid = lax.axis_index("s")
    wid = sid * NC + lax.axis_index("c")
    base = wid * TOK_PER_W

    idx_cp = pltpu.async_copy(tok_hbm.at[wid], idx_v, isem)
    pltpu.sync_copy(table_hbm, tab_v)
    pltpu.sync_copy(w_hbm, w_v)
    pltpu.sync_copy(b_hbm, b_v)

    tab = []
    for k in range(K):
        trow = tab_v[k, pl.ds(0, VL)]
        tab.append([trow[i] for i in range(NIN)])
    for c in range(D // VL):
        bvec = b_v[pl.ds(c * VL, VL)]
        wcol = [w_v[i, pl.ds(c * VL, VL)] for i in range(NIN)]
        for k in range(K):
            acc = bvec
            for i in range(NIN):
                acc = acc + tab[k][i] * wcol[i]
            pv[k, pl.ds(c * VL, VL)] = acc
    psrc = pshared.at[pl.ds(sid * K, K)]
    plsc.subcore_barrier()
    pltpu.sync_copy(pv, psrc)
    plsc.subcore_barrier()
    idx_cp.wait()

    gd = [None] * NCHUNK
    wd = [None] * NCHUNK

    def write_back(i):
        b = i % NBUF
        gd[i].wait()
        wd[i] = pltpu.async_copy(
            rows[b], out_hbm.at[pl.ds(base + i * CHUNK, CHUNK)], wsem[b]
        )

    for j in range(NCHUNK):
        b = j % NBUF
        if j >= NBUF:
            wd[j - NBUF].wait()
        gd[j] = pltpu.async_copy(psrc.at[idx_v.at[j]], rows[b], gsem[b])
        if j >= LAG:
            write_back(j - LAG)
    for i in range(NCHUNK - LAG, NCHUNK):
        write_back(i)
    for i in range(NCHUNK - NBUF, NCHUNK):
        wd[i].wait()


def kernel(tokens, table, W, b):
    tok3 = tokens.astype(jnp.int32).reshape(NW, NCHUNK, CHUNK)
    tab16 = jnp.pad(table, ((0, 0), (0, VL - NIN)))
    return _sc_embed(tab16, W, b, tok3)
